# baseline (speedup 1.0000x reference)
import functools

import jax
import jax.numpy as jnp
from jax import lax
from jax.experimental import pallas as pl
from jax.experimental.pallas import tpu as pltpu
from jax.experimental.pallas import tpu_sc as plsc

N = 1_000_000
LANES = 16
NTILES = 16
CHUNK = 62_496
NVREG = CHUNK // LANES
TAIL_BASE = NTILES * CHUNK
TAIL_TILES = (N - TAIL_BASE) // LANES

I32_MAX = 2**31 - 1
F32_INF = float("inf")


def _body(in_hbm, p_hbm, out_hbm,
          chunk, extra, pvm, pd, pi, pv, shd, shi, shv, alld, alli, allv):
    w = lax.axis_index("s")
    base = w * CHUNK

    pltpu.sync_copy(p_hbm, pvm)
    pltpu.sync_copy(in_hbm.at[pl.ds(base, CHUNK)], chunk)
    has_tail = w < TAIL_TILES
    e_off = TAIL_BASE + jnp.where(has_tail, w, 0) * LANES
    pltpu.sync_copy(in_hbm.at[pl.ds(e_off, LANES)], extra)

    pvec = pvm[...]
    iota = lax.iota(jnp.int32, LANES)

    bd0 = jnp.full((LANES,), F32_INF, jnp.float32)
    bi0 = jnp.zeros((LANES,), jnp.int32)
    bv0 = jnp.zeros((LANES,), jnp.float32)
    idx0 = iota + base

    def step(j, carry):
        bd, bi, bv, idxv = carry
        x = chunk[pl.ds(j * LANES, LANES)]
        d = jnp.abs(x - pvec)
        m = d < bd
        bd = jnp.where(m, d, bd)
        bi = jnp.where(m, idxv, bi)
        bv = jnp.where(m, x, bv)
        return bd, bi, bv, idxv + LANES

    bd, bi, bv, _ = lax.fori_loop(0, NVREG, step, (bd0, bi0, bv0, idx0))

    xt = extra[...]
    dt = jnp.abs(xt - pvec)
    dt = jnp.where(jnp.full((LANES,), has_tail), dt, F32_INF)
    it = iota + e_off
    mt = dt < bd
    bd = jnp.where(mt, dt, bd)
    bi = jnp.where(mt, it, bi)
    bv = jnp.where(mt, xt, bv)

    pd[...] = bd
    pi[...] = bi
    pv[...] = bv
    pltpu.sync_copy(pd, shd.at[w])
    pltpu.sync_copy(pi, shi.at[w])
    pltpu.sync_copy(pv, shv.at[w])
    plsc.subcore_barrier()

    @pl.when(w == 0)
    def _():
        pltpu.sync_copy(shd, alld)
        pltpu.sync_copy(shi, alli)
        pltpu.sync_copy(shv, allv)
        gd = alld[0]
        gi = alli[0]
        gv = allv[0]
        for r in range(1, NTILES):
            rd = alld[r]
            ri = alli[r]
            rv = allv[r]
            better = (rd < gd) | ((rd == gd) & (ri < gi))
            gd = jnp.where(better, rd, gd)
            gi = jnp.where(better, ri, gi)
            gv = jnp.where(better, rv, gv)
        for s in (8, 4, 2, 1):
            pd[...] = gd
            pi[...] = gi
            pv[...] = gv
            perm = (iota + s) & (LANES - 1)
            rd = plsc.load_gather(pd, [perm])
            ri = plsc.load_gather(pi, [perm])
            rv = plsc.load_gather(pv, [perm])
            better = (rd < gd) | ((rd == gd) & (ri < gi))
            gd = jnp.where(better, rd, gd)
            gi = jnp.where(better, ri, gi)
            gv = jnp.where(better, rv, gv)
        pv[...] = gv
        pltpu.sync_copy(pv, out_hbm)


@jax.jit
def _closest_sc(inp, p16):
    mesh = plsc.VectorSubcoreMesh(
        core_axis_name="c", subcore_axis_name="s", num_cores=1)
    f = pl.kernel(
        _body,
        out_type=jax.ShapeDtypeStruct((LANES,), jnp.float32),
        mesh=mesh,
        compiler_params=pltpu.CompilerParams(needs_layout_passes=False, use_tc_tiling_on_sc=False),
        scratch_types=[
            pltpu.VMEM((CHUNK,), jnp.float32),
            pltpu.VMEM((LANES,), jnp.float32),
            pltpu.VMEM((LANES,), jnp.float32),
            pltpu.VMEM((LANES,), jnp.float32),
            pltpu.VMEM((LANES,), jnp.int32),
            pltpu.VMEM((LANES,), jnp.float32),
            pltpu.VMEM_SHARED((NTILES, LANES), jnp.float32),
            pltpu.VMEM_SHARED((NTILES, LANES), jnp.int32),
            pltpu.VMEM_SHARED((NTILES, LANES), jnp.float32),
            pltpu.VMEM((NTILES, LANES), jnp.float32),
            pltpu.VMEM((NTILES, LANES), jnp.int32),
            pltpu.VMEM((NTILES, LANES), jnp.float32),
        ],
    )
    return f(inp, p16)


def kernel(input, prev_output):
    p16 = jnp.broadcast_to(prev_output, (LANES,))
    out16 = _closest_sc(input, p16)
    return out16[:1]


# speedup vs baseline: 1.2803x; 1.2803x over previous
import jax
import jax.numpy as jnp
from jax import lax
from jax.experimental import pallas as pl
from jax.experimental.pallas import tpu as pltpu
from jax.experimental.pallas import tpu_sc as plsc

N = 1_000_000
LANES = 16
NTILES = 16
CHUNK = 62_496
NVREG = CHUNK // LANES
TAIL_BASE = NTILES * CHUNK
TAIL_TILES = (N - TAIL_BASE) // LANES
UNROLL = 6
SLICES = (978, 978, 978, 972)
assert sum(SLICES) == NVREG and all(s % UNROLL == 0 for s in SLICES)

I32_MAX = 2**31 - 1
SIGN_MASK = 0x7FFFFFFF
F32_INF = float("inf")


def _body(in_hbm, p_hbm, out_hbm,
          chunk, pvm, pd, pi, pv, shd, shi, shv, alld, alli, allv,
          sem0, sem1, sem2, sem3, sem4):
    w = lax.axis_index("s")
    base = w * CHUNK

    pltpu.sync_copy(p_hbm, pvm)
    sems = (sem0, sem1, sem2, sem3)
    copies = []
    off = 0
    for q, nv in enumerate(SLICES):
        ne = nv * LANES
        copies.append(pltpu.async_copy(
            in_hbm.at[pl.ds(base + off, ne)], chunk.at[pl.ds(off, ne)],
            sems[q]))
        off += ne
    has_tail = w < TAIL_TILES
    e_off = TAIL_BASE + jnp.where(has_tail, w, 0) * LANES
    tail_copy = pltpu.async_copy(
        in_hbm.at[pl.ds(e_off, LANES)], chunk.at[pl.ds(CHUNK, LANES)], sem4)

    pvec = pvm[...]
    iota = lax.iota(jnp.int32, LANES)

    accs = [(jnp.full((LANES,), I32_MAX, jnp.int32),
             jnp.zeros((LANES,), jnp.int32)) for _ in range(UNROLL)]
    idxs = [iota + k * LANES for k in range(UNROLL)]
    carry0 = tuple(a for pair in accs for a in pair) + tuple(idxs)

    off = 0
    carry = carry0
    for q, nv in enumerate(SLICES):
        copies[q].wait()

        def step(j, c, off=off):
            c = list(c)
            for k in range(UNROLL):
                bd, bi, idxv = c[2 * k], c[2 * k + 1], c[2 * UNROLL + k]
                x = chunk[pl.ds(off + (j * UNROLL + k) * LANES, LANES)]
                xi = lax.bitcast_convert_type(x - pvec, jnp.int32) & SIGN_MASK
                m = xi < bd
                c[2 * k] = jnp.where(m, xi, bd)
                c[2 * k + 1] = jnp.where(m, idxv, bi)
                c[2 * UNROLL + k] = idxv + UNROLL * LANES
            return tuple(c)

        carry = lax.fori_loop(0, nv // UNROLL, step, carry)
        off += nv * LANES

    tail_copy.wait()
    xt = chunk[pl.ds(CHUNK, LANES)]
    ti = lax.bitcast_convert_type(xt - pvec, jnp.int32) & SIGN_MASK
    ti = jnp.where(jnp.full((LANES,), has_tail), ti, I32_MAX)
    it = jnp.full((LANES,), CHUNK, jnp.int32) + iota
    bd, bi = carry[0], carry[1]
    mt = ti < bd
    bd = jnp.where(mt, ti, bd)
    bi = jnp.where(mt, it, bi)

    for k in range(1, UNROLL):
        rd, ri = carry[2 * k], carry[2 * k + 1]
        better = (rd < bd) | ((rd == bd) & (ri < bi))
        bd = jnp.where(better, rd, bd)
        bi = jnp.where(better, ri, bi)

    bv = plsc.load_gather(chunk, [bi])
    in_main = bi < CHUNK
    bg = jnp.where(in_main, bi + base, bi - CHUNK + (TAIL_BASE + w * LANES))

    pd[...] = bd
    pi[...] = bg
    pv[...] = bv
    pltpu.sync_copy(pd, shd.at[w])
    pltpu.sync_copy(pi, shi.at[w])
    pltpu.sync_copy(pv, shv.at[w])
    plsc.subcore_barrier()

    @pl.when(w == 0)
    def _():
        pltpu.sync_copy(shd, alld)
        pltpu.sync_copy(shi, alli)
        pltpu.sync_copy(shv, allv)
        gd = alld[0]
        gi = alli[0]
        gv = allv[0]
        for r in range(1, NTILES):
            rd = alld[r]
            ri = alli[r]
            rv = allv[r]
            better = (rd < gd) | ((rd == gd) & (ri < gi))
            gd = jnp.where(better, rd, gd)
            gi = jnp.where(better, ri, gi)
            gv = jnp.where(better, rv, gv)
        for s in (8, 4, 2, 1):
            pd[...] = gd
            pi[...] = gi
            pv[...] = gv
            perm = (iota + s) & (LANES - 1)
            rd = plsc.load_gather(pd, [perm])
            ri = plsc.load_gather(pi, [perm])
            rv = plsc.load_gather(pv, [perm])
            better = (rd < gd) | ((rd == gd) & (ri < gi))
            gd = jnp.where(better, rd, gd)
            gi = jnp.where(better, ri, gi)
            gv = jnp.where(better, rv, gv)
        pv[...] = gv
        pltpu.sync_copy(pv, out_hbm)


@jax.jit
def _closest_sc(inp, p16):
    mesh = plsc.VectorSubcoreMesh(
        core_axis_name="c", subcore_axis_name="s", num_cores=1)
    f = pl.kernel(
        _body,
        out_type=jax.ShapeDtypeStruct((LANES,), jnp.float32),
        mesh=mesh,
        compiler_params=pltpu.CompilerParams(
            needs_layout_passes=False, use_tc_tiling_on_sc=False),
        scratch_types=[
            pltpu.VMEM((CHUNK + LANES,), jnp.float32),
            pltpu.VMEM((LANES,), jnp.float32),
            pltpu.VMEM((LANES,), jnp.int32),
            pltpu.VMEM((LANES,), jnp.int32),
            pltpu.VMEM((LANES,), jnp.float32),
            pltpu.VMEM_SHARED((NTILES, LANES), jnp.int32),
            pltpu.VMEM_SHARED((NTILES, LANES), jnp.int32),
            pltpu.VMEM_SHARED((NTILES, LANES), jnp.float32),
            pltpu.VMEM((NTILES, LANES), jnp.int32),
            pltpu.VMEM((NTILES, LANES), jnp.int32),
            pltpu.VMEM((NTILES, LANES), jnp.float32),
            pltpu.SemaphoreType.DMA,
            pltpu.SemaphoreType.DMA,
            pltpu.SemaphoreType.DMA,
            pltpu.SemaphoreType.DMA,
            pltpu.SemaphoreType.DMA,
        ],
    )
    return f(inp, p16)


def kernel(input, prev_output):
    p16 = jnp.broadcast_to(prev_output, (LANES,))
    out16 = _closest_sc(input, p16)
    return out16[:1]
